# baseline (speedup 1.0000x reference)
import functools

import jax
import jax.numpy as jnp
from jax import lax
from jax.experimental import pallas as pl
from jax.experimental.pallas import tpu as pltpu
from jax.experimental.pallas import tpu_sc as plsc

_USER_LEN = 1000
_L = 200
_NROW = _L + 2
_NPAD = 208
_HS = 128
_NW = 13
_FIN = 15


def _fm_body(cidx_hbm, cidx_hi_hbm, itemrow_hbm, user_hbm, extras_hbm,
             feature_hbm, bias2d_hbm,
             out_nz, out_b, out_res,
             idx_v, hi_v, buf, buf_bias, pbuf, ext_v, bstage,
             res_v, fin_buf, shared, sem, sem2):
    c = lax.axis_index("c")
    s = lax.axis_index("s")
    lane_iota = lax.iota(jnp.int32, 16)

    @pl.when(jnp.logical_and(c == 0, s < _NW))
    def _():
        base = pl.multiple_of(16 * s, 16)
        pltpu.sync_copy(cidx_hbm.at[pl.ds(base, 16)], idx_v)
        pltpu.sync_copy(cidx_hi_hbm.at[pl.ds(base, 16)], hi_v)
        cpA = pltpu.async_copy(
            feature_hbm.at[idx_v, pl.ds(0, _HS)], buf, sem)
        cpB = pltpu.async_copy(bias2d_hbm.at[hi_v], buf_bias, sem2)
        cpA.wait()

        @pl.when(s == 0)
        def _():
            pltpu.sync_copy(itemrow_hbm.at[:, pl.ds(0, _HS)],
                            buf.at[pl.ds(1, 1), :])
            pltpu.sync_copy(user_hbm.at[:, pl.ds(0, _HS)],
                            buf.at[pl.ds(0, 1), :])
            pltpu.sync_copy(extras_hbm, ext_v.at[pl.ds(0, 8)])
            pltpu.sync_copy(buf.at[pl.ds(0, 2), :],
                            shared.at[pl.ds(_NW, 2), :])

        @pl.when(s < _NW - 1)
        def _():
            pltpu.sync_copy(buf, out_nz.at[pl.ds(base, 16), :])

        @pl.when(s == _NW - 1)
        def _():
            pltpu.sync_copy(buf.at[pl.ds(0, 10), :],
                            out_nz.at[pl.ds(192, 10), :])

        cpB.wait()
        ivec = idx_v[...]
        lanes = jnp.bitwise_and(ivec, 127)
        vals = plsc.load_gather(buf_bias, [lane_iota, lanes])

        @pl.when(s == 0)
        def _():
            ev = ext_v[...]
            bstage[...] = jnp.where(lane_iota < 2, ev, vals)

        @pl.when(s != 0)
        def _():
            bstage[...] = vals

        @pl.when(s < _NW - 1)
        def _():
            pltpu.sync_copy(bstage, out_b.at[pl.ds(base, 16)])

        @pl.when(s == _NW - 1)
        def _():
            pltpu.sync_copy(bstage.at[pl.ds(0, 10)],
                            out_b.at[pl.ds(192, 10)])

        lo = jnp.where(s == 0, 2, 0)
        hi = jnp.where(s == _NW - 1, 10, 16)

        def body(r, acc):
            return tuple(acc[j] + buf[r, pl.ds(16 * j, 16)] for j in range(8))

        acc0 = tuple(jnp.zeros((16,), jnp.float32) for _ in range(8))
        colsum = lax.fori_loop(lo, hi, body, acc0)
        for j in range(8):
            pbuf[0, pl.ds(16 * j, 16)] = colsum[j]
        pltpu.sync_copy(pbuf, shared.at[pl.ds(s, 1), :])

    plsc.subcore_barrier()

    @pl.when(jnp.logical_and(c == 0, s == _FIN))
    def _():
        pltpu.sync_copy(shared.at[pl.ds(0, _NW + 2), :], fin_buf)
        pltpu.sync_copy(extras_hbm, ext_v.at[pl.ds(0, 8)])

        def body(r, acc):
            return tuple(acc[j] + fin_buf[r, pl.ds(16 * j, 16)]
                         for j in range(8))

        acc0 = tuple(jnp.zeros((16,), jnp.float32) for _ in range(8))
        colsum = lax.fori_loop(0, _NW, body, acc0)

        t = jnp.zeros((16,), jnp.float32)
        for j in range(8):
            u = fin_buf[_NW, pl.ds(16 * j, 16)]
            iv = fin_buf[_NW + 1, pl.ds(16 * j, 16)]
            t = t + u * iv + (u + iv) * colsum[j]
        total = t[0]
        for lane in range(1, 16):
            total = total + t[lane]
        ev = ext_v[...]
        res_v[...] = jnp.zeros((16,), jnp.float32) + (ev[2] + total)
        pltpu.sync_copy(res_v.at[pl.ds(0, 1)], out_res)


_fm_gather = functools.partial(
    pl.kernel,
    mesh=plsc.VectorSubcoreMesh(core_axis_name="c", subcore_axis_name="s",
                                num_cores=1),
    compiler_params=pltpu.CompilerParams(needs_layout_passes=False),
    out_type=[
        jax.ShapeDtypeStruct((_NROW, _HS), jnp.float32),
        jax.ShapeDtypeStruct((_NROW,), jnp.float32),
        jax.ShapeDtypeStruct((1,), jnp.float32),
    ],
    scratch_types=[
        pltpu.VMEM((16,), jnp.int32),
        pltpu.VMEM((16,), jnp.int32),
        pltpu.VMEM((16, _HS), jnp.float32),
        pltpu.VMEM((16, _HS), jnp.float32),
        pltpu.VMEM((1, _HS), jnp.float32),
        pltpu.VMEM((16,), jnp.float32),
        pltpu.VMEM((16,), jnp.float32),
        pltpu.VMEM((16,), jnp.float32),
        pltpu.VMEM((_NW + 2, _HS), jnp.float32),
        pltpu.VMEM_SHARED((_NW + 2, _HS), jnp.float32),
        pltpu.SemaphoreType.DMA,
        pltpu.SemaphoreType.DMA,
    ],
)(_fm_body)


def kernel(items_emb, feature_emb, user_emb, Bias, ui_pair, feature_index,
           preference_index):
    del feature_index
    pref_idx = preference_index.reshape(_L).astype(jnp.int32)
    cidx = jnp.concatenate(
        [jnp.zeros((2,), jnp.int32), pref_idx,
         jnp.zeros((_NPAD - _NROW,), jnp.int32)])
    cidx_hi = jnp.right_shift(cidx, 7)
    item_idx = (ui_pair[0, 1:2].astype(jnp.int32) - _USER_LEN)
    item_row = jnp.take(items_emb, item_idx, axis=0)
    bias2d = jnp.pad(feature_emb[:, _HS], (0, 96)).reshape(-1, _HS)
    extras = jnp.concatenate(
        [user_emb[0:1, _HS], item_row[0:1, _HS],
         Bias.astype(jnp.float32), jnp.zeros((5,), jnp.float32)])
    out_nz, out_b, out_res = _fm_gather(
        cidx, cidx_hi, item_row, user_emb, extras,
        feature_emb, bias2d)
    return (out_res.reshape(1, 1),
            out_b.reshape(1, _NROW, 1),
            out_nz.reshape(1, _NROW, _HS))


# speedup vs baseline: 1.1684x; 1.1684x over previous
import functools

import jax
import jax.numpy as jnp
from jax import lax
from jax.experimental import pallas as pl
from jax.experimental.pallas import tpu as pltpu
from jax.experimental.pallas import tpu_sc as plsc

_USER_LEN = 1000
_L = 200
_NROW = _L + 2
_NPAD = 208
_HS = 128
_NV = 100000
_NVA = 99968
_CH = 8192
_NFULL = _NVA // _CH
_REST = _NVA - _NFULL * _CH
_NTAIL = _NV - _NVA
_NVEC = _NPAD // 16


def _fm_body(cidx_hbm, itemrow_hbm, user_hbm, extras_hbm, tabt_hbm,
             tail_hbm,
             out_t, out_res,
             idx_all, ubuf, ibuf, ext_v, chunk, ostage, pbuf, res_v,
             fin_buf, tail_v, shared, sem):
    c = lax.axis_index("c")
    s = lax.axis_index("s")
    lane_iota = lax.iota(jnp.int32, 16)
    zero16 = jnp.zeros((16,), jnp.int32)

    @pl.when(c == 0)
    def _():
        pltpu.sync_copy(cidx_hbm, idx_all)
        pltpu.sync_copy(user_hbm, ubuf)
        pltpu.sync_copy(itemrow_hbm, ibuf)
        pltpu.sync_copy(tail_hbm, tail_v)
        pltpu.sync_copy(extras_hbm, ext_v.at[pl.ds(0, 8)])
        ev = ext_v[...]
        idxv = [idx_all[pl.ds(16 * t, 16)] for t in range(_NVEC)]
        rblock = pl.multiple_of(8 * s, 8)

        def sweep(lo, size, rows):
            for j in range(rows):
                for t in range(_NVEC):
                    rel = idxv[t] - lo
                    msk = jnp.logical_and(idxv[t] >= lo,
                                          idxv[t] < lo + size)
                    g = plsc.load_gather(
                        chunk, [zero16 + j, jnp.where(msk, rel, 0)])
                    prev = ostage[j, pl.ds(16 * t, 16)]
                    ostage[j, pl.ds(16 * t, 16)] = jnp.where(msk, g, prev)

        def chunk_body(cc, carry):
            off = pl.multiple_of(cc * _CH, 128)
            pltpu.sync_copy(
                tabt_hbm.at[pl.ds(rblock, 8), pl.ds(off, _CH)], chunk)
            sweep(cc * _CH, _CH, 8)
            return carry

        lax.fori_loop(0, _NFULL, chunk_body, jnp.int32(0))
        pltpu.sync_copy(
            tabt_hbm.at[pl.ds(rblock, 8), pl.ds(_NFULL * _CH, _REST)],
            chunk.at[:, pl.ds(0, _REST)])
        sweep(_NFULL * _CH, _REST, 8)

        @pl.when(s == 15)
        def _():
            def bias_body(cc, carry):
                off = pl.multiple_of(cc * _CH, 128)
                pltpu.sync_copy(
                    tabt_hbm.at[pl.ds(128, 1), pl.ds(off, _CH)],
                    chunk.at[pl.ds(0, 1), :])
                lo = cc * _CH
                for t in range(_NVEC):
                    rel = idxv[t] - lo
                    msk = jnp.logical_and(idxv[t] >= lo, idxv[t] < lo + _CH)
                    g = plsc.load_gather(
                        chunk, [zero16, jnp.where(msk, rel, 0)])
                    prev = ostage[8, pl.ds(16 * t, 16)]
                    ostage[8, pl.ds(16 * t, 16)] = jnp.where(msk, g, prev)
                return carry

            lax.fori_loop(0, _NFULL, bias_body, jnp.int32(0))
            pltpu.sync_copy(
                tabt_hbm.at[pl.ds(128, 1), pl.ds(_NFULL * _CH, _REST)],
                chunk.at[pl.ds(0, 1), pl.ds(0, _REST)])
            lo = _NFULL * _CH
            for t in range(_NVEC):
                rel = idxv[t] - lo
                msk = jnp.logical_and(idxv[t] >= lo, idxv[t] < _NVA)
                g = plsc.load_gather(
                    chunk, [zero16, jnp.where(msk, rel, 0)])
                mskT = idxv[t] >= _NVA
                gT = plsc.load_gather(
                    tail_v, [jnp.where(mskT, idxv[t] - _NVA, 0),
                             zero16 + _HS])
                prev = ostage[8, pl.ds(16 * t, 16)]
                ostage[8, pl.ds(16 * t, 16)] = jnp.where(
                    mskT, gT, jnp.where(msk, g, prev))
            b0 = ostage[8, pl.ds(0, 16)]
            ostage[8, pl.ds(0, 16)] = jnp.where(
                lane_iota < 1, ev[0], jnp.where(lane_iota < 2, ev[1], b0))

        m_first = (lane_iota >= 2).astype(jnp.float32)
        m_last = (lane_iota < 10).astype(jnp.float32)
        part = jnp.float32(0.0)
        for j in range(8):
            cpos = zero16 + (8 * s + j)
            for t in range(_NVEC):
                mskT = idxv[t] >= _NVA
                gT = plsc.load_gather(
                    tail_v, [jnp.where(mskT, idxv[t] - _NVA, 0), cpos])
                prev = ostage[j, pl.ds(16 * t, 16)]
                ostage[j, pl.ds(16 * t, 16)] = jnp.where(mskT, gT, prev)
            u_c = plsc.load_gather(ubuf, [zero16, cpos])[0]
            i_c = plsc.load_gather(ibuf, [zero16, cpos])[0]
            a0 = ostage[j, pl.ds(0, 16)]
            a0 = jnp.where(lane_iota < 1, u_c,
                           jnp.where(lane_iota < 2, i_c, a0))
            ostage[j, pl.ds(0, 16)] = a0
            ssum = a0 * m_first
            for t in range(1, _NVEC - 1):
                ssum = ssum + ostage[j, pl.ds(16 * t, 16)]
            ssum = ssum + ostage[j, pl.ds(16 * (_NVEC - 1), 16)] * m_last
            s_c = ssum[0]
            for lane in range(1, 16):
                s_c = s_c + ssum[lane]
            part = part + (u_c * i_c + (u_c + i_c) * s_c)

        pbuf[0, pl.ds(0, 16)] = jnp.zeros((16,), jnp.float32) + part
        for jj in range(1, 8):
            pbuf[0, pl.ds(16 * jj, 16)] = jnp.zeros((16,), jnp.float32)
        pltpu.sync_copy(pbuf, shared.at[pl.ds(s, 1), :])

        @pl.when(s < 15)
        def _():
            pltpu.sync_copy(ostage.at[pl.ds(0, 8), :],
                            out_t.at[pl.ds(rblock, 8), :])

        @pl.when(s == 15)
        def _():
            pltpu.sync_copy(ostage, out_t.at[pl.ds(120, 16), :])

    plsc.subcore_barrier()

    @pl.when(jnp.logical_and(c == 0, s == 15))
    def _():
        pltpu.sync_copy(shared, fin_buf)

        def body(r, tot):
            v = fin_buf[r, pl.ds(0, 16)]
            return tot + v[0]

        total = lax.fori_loop(0, 16, body, jnp.float32(0.0))
        ev = ext_v[...]
        res_v[...] = jnp.zeros((16,), jnp.float32) + (ev[2] + total)
        pltpu.sync_copy(res_v.at[pl.ds(0, 1)], out_res)


_fm_gather = functools.partial(
    pl.kernel,
    mesh=plsc.VectorSubcoreMesh(core_axis_name="c", subcore_axis_name="s",
                                num_cores=1),
    compiler_params=pltpu.CompilerParams(needs_layout_passes=False),
    out_type=[
        jax.ShapeDtypeStruct((136, _NPAD), jnp.float32),
        jax.ShapeDtypeStruct((1,), jnp.float32),
    ],
    scratch_types=[
        pltpu.VMEM((_NPAD,), jnp.int32),
        pltpu.VMEM((1, _HS + 1), jnp.float32),
        pltpu.VMEM((1, _HS + 1), jnp.float32),
        pltpu.VMEM((16,), jnp.float32),
        pltpu.VMEM((8, _CH), jnp.float32),
        pltpu.VMEM((16, _NPAD), jnp.float32),
        pltpu.VMEM((1, _HS), jnp.float32),
        pltpu.VMEM((16,), jnp.float32),
        pltpu.VMEM((16, _HS), jnp.float32),
        pltpu.VMEM((_NTAIL, _HS + 1), jnp.float32),
        pltpu.VMEM_SHARED((16, _HS), jnp.float32),
        pltpu.SemaphoreType.DMA,
    ],
)(_fm_body)


def kernel(items_emb, feature_emb, user_emb, Bias, ui_pair, feature_index,
           preference_index):
    del feature_index
    pref_idx = preference_index.reshape(_L).astype(jnp.int32)
    cidx = jnp.concatenate(
        [jnp.zeros((2,), jnp.int32), pref_idx,
         jnp.zeros((_NPAD - _NROW,), jnp.int32)])
    item_idx = (ui_pair[0, 1:2].astype(jnp.int32) - _USER_LEN)
    item_row = jnp.take(items_emb, item_idx, axis=0)
    extras = jnp.concatenate(
        [user_emb[0:1, _HS], item_row[0:1, _HS],
         Bias.astype(jnp.float32), jnp.zeros((5,), jnp.float32)])
    tabt = feature_emb.T
    tail_rows = feature_emb[_NVA:_NV, :]
    out_t, out_res = _fm_gather(cidx, item_row, user_emb, extras, tabt,
                                tail_rows)
    nz = out_t[:_HS, :_NROW].T
    bias_col = out_t[_HS, :_NROW]
    return (out_res.reshape(1, 1),
            bias_col.reshape(1, _NROW, 1),
            nz.reshape(1, _NROW, _HS))
